# baseline (speedup 1.0000x reference)
import functools

import jax
import jax.numpy as jnp
from jax import lax
from jax.experimental import pallas as pl
from jax.experimental.pallas import tpu as pltpu
from jax.experimental.pallas import tpu_sc as plsc

S = 512
B = 32
C3 = 3
PLANES = B * C3
RCH = 32
NCHUNK = S // RCH
ITEMS = PLANES * NCHUNK
NWORK = 32
PER_W = ITEMS // NWORK
LANES = 16


def _copy_sum_body(x_ref, out_ref, cs_ref):
    x = x_ref[...]
    out_ref[...] = x
    cs_ref[0, 0, :] = jnp.sum(x[0, 0], axis=0)


def _pass_a(image):
    return pl.pallas_call(
        _copy_sum_body,
        grid=(B, C3),
        in_specs=[pl.BlockSpec((1, 1, S, S), lambda b, c: (b, c, 0, 0))],
        out_specs=[
            pl.BlockSpec((1, 1, S, S), lambda b, c: (b, c, 0, 0)),
            pl.BlockSpec((1, 1, S), lambda b, c: (b * C3 + c, 0, 0)),
        ],
        out_shape=[
            jax.ShapeDtypeStruct((B, C3, S, S), jnp.float32),
            jax.ShapeDtypeStruct((PLANES, 1, S), jnp.float32),
        ],
    )(image)


def _color_body(cs_ref, ann_ref, w_ref, b_ref, fparams_ref, ibounds_ref):
    cs = cs_ref[...]
    sums = jnp.sum(cs[:, 0, :], axis=1)
    pooled = sums.reshape(B, C3) * (1.0 / (S * S))
    feat = jnp.dot(pooled, w_ref[...],
                   preferred_element_type=jnp.float32) + b_ref[...]
    sig = 1.0 / (1.0 + jnp.exp(-feat))
    color = sig[:, :3]
    t = sig[:, 3:4]
    fparams = jnp.concatenate(
        [color * (1.0 - t), t, jnp.zeros((B, 12), jnp.float32)], axis=1)
    fparams_ref[...] = fparams

    ann = jnp.clip(ann_ref[...], 0.0, 1.0) * S
    x1 = ann[:, 0:1]
    y1 = ann[:, 1:2]
    x2 = x1 + ann[:, 2:3]
    y2 = y1 + ann[:, 3:4]
    xlo = jnp.floor(x1) + 1.0
    ylo = jnp.floor(y1) + 1.0
    xhi = jnp.minimum(jnp.ceil(x2) - 1.0, S - 1.0)
    yhi = jnp.minimum(jnp.ceil(y2) - 1.0, S - 1.0)
    bounds = jnp.concatenate([ylo, yhi, xlo, xhi], axis=1).astype(jnp.int32)
    ibounds_ref[...] = jnp.concatenate(
        [bounds, jnp.zeros((B, 12), jnp.int32)], axis=1)


def _pass_c(colsums, annotations, W_color, b_color):
    return pl.pallas_call(
        _color_body,
        in_specs=[pl.BlockSpec((PLANES, 1, S), lambda: (0, 0, 0)),
                  pl.BlockSpec((B, 4), lambda: (0, 0)),
                  pl.BlockSpec((C3, 4), lambda: (0, 0)),
                  pl.BlockSpec((1, 4), lambda: (0, 0))],
        out_specs=[pl.BlockSpec((B, 16), lambda: (0, 0)),
                   pl.BlockSpec((B, 16), lambda: (0, 0))],
        out_shape=[jax.ShapeDtypeStruct((B, 16), jnp.float32),
                   jax.ShapeDtypeStruct((B, 16), jnp.int32)],
    )(colsums, annotations, W_color, b_color.reshape(1, 4))


def _sc_blend(img_ref, fparams_hbm, ibounds_hbm,
              fparams_v, ibounds_v, chunk_v):
    pltpu.sync_copy(fparams_hbm, fparams_v)
    pltpu.sync_copy(ibounds_hbm, ibounds_v)

    wid = lax.axis_index("s") * 2 + lax.axis_index("c")

    def per_item(i, _):
        item = wid + NWORK * i
        b = item // (C3 * NCHUNK)
        rem = item % (C3 * NCHUNK)
        c = rem // NCHUNK
        k = rem % NCHUNK
        r0 = k * RCH

        iv = ibounds_v[b, pl.ds(0, LANES)]
        ylo = iv[0]
        yhi = iv[1]
        xlo = iv[2]
        xhi = iv[3]
        rlo = jnp.maximum(r0, ylo)
        rhi = jnp.minimum(r0 + RCH - 1, yhi)

        @pl.when(jnp.logical_and(rlo <= rhi, xlo <= xhi))
        def _():
            pltpu.sync_copy(img_ref.at[b, c, pl.ds(r0, RCH), :], chunk_v)
            fv = fparams_v[b, pl.ds(0, LANES)]
            cval = jnp.where(c == 0, fv[0], jnp.where(c == 1, fv[1], fv[2]))
            t = fv[3]

            def per_colblock(jb, _):
                j0 = jb * LANES
                jv = lax.iota(jnp.int32, LANES) + j0
                mask = jnp.logical_and(jv >= xlo, jv <= xhi)

                def per_row(r, _):
                    ri = r - r0
                    v = chunk_v[ri, pl.ds(j0, LANES)]
                    chunk_v[ri, pl.ds(j0, LANES)] = jnp.where(
                        mask, cval + t * v, v)
                    return 0

                lax.fori_loop(rlo, rhi + 1, per_row, 0)
                return 0

            lax.fori_loop(xlo // LANES, xhi // LANES + 1, per_colblock, 0)
            pltpu.sync_copy(chunk_v, img_ref.at[b, c, pl.ds(r0, RCH), :])

        return 0

    lax.fori_loop(0, PER_W, per_item, 0)


def _make_sc_kernel():
    mesh = plsc.VectorSubcoreMesh(
        core_axis_name="c", subcore_axis_name="s",
        num_cores=2, num_subcores=16)
    return pl.kernel(
        _sc_blend,
        out_type=(),
        mesh=mesh,
        scratch_types=[
            pltpu.VMEM((B, 16), jnp.float32),
            pltpu.VMEM((B, 16), jnp.int32),
            pltpu.VMEM((RCH, S), jnp.float32),
        ],
    )


def kernel(image, annotations, W_color, b_color):
    copy, colsums = _pass_a(image)
    fparams, ibounds = _pass_c(colsums, annotations, W_color, b_color)
    ref = jax.new_ref(copy)
    _make_sc_kernel()(ref, fparams, ibounds)
    return ref[...]
